# Initial kernel scaffold; baseline (speedup 1.0000x reference)
import jax
import jax.numpy as jnp
from jax.experimental import pallas as pl


def kernel(x, edge_index, W1_l, b1_l, W1_r, W2_l, b2_l, W2_r):
    raise NotImplementedError("write your pallas kernel here")


# speedup vs baseline: 2.3336x; 2.3336x over previous
import functools

import jax
import jax.numpy as jnp
from jax import lax
from jax.experimental import pallas as pl
from jax.experimental.pallas import tpu as pltpu
from jax.experimental.pallas import tpu_sc as plsc

N = 10000
E = 320000
D = 128
NC = 2
NS = 16
CHUNK = 64
PT = 10240
STEPS = PT // CHUNK
EPAD = NC * NS * PT
NPAD = 10240
STRIPE = NPAD // NS


def _zero_acc(acc_sh, rows_v, idx_lin, lin_hbm, s):
  for j in range(STRIPE // CHUNK):
    pltpu.sync_copy(lin_hbm.at[pl.ds(s * STRIPE + j * CHUNK, CHUNK)], idx_lin)
    pltpu.sync_copy(rows_v, acc_sh.at[idx_lin])


def _copy_out(acc_sh, rows_v, idx_lin, lin_hbm, out_hbm, c, s, sem):
  for j in range(STRIPE // CHUNK):
    r0 = s * STRIPE + j * CHUNK
    pltpu.sync_copy(lin_hbm.at[pl.ds(r0, CHUNK)], idx_lin)
    pltpu.async_copy(acc_sh.at[idx_lin], rows_v, sem).wait()
    pltpu.sync_copy(rows_v, out_hbm.at[pl.ds(c * NPAD + r0, CHUNK)])


_SC_SCRATCH = [
    pltpu.VMEM_SHARED((NPAD, D), jnp.float32),
    pltpu.VMEM((CHUNK,), jnp.int32),
    pltpu.VMEM((CHUNK,), jnp.int32),
    pltpu.VMEM((CHUNK,), jnp.int32),
    pltpu.VMEM((CHUNK, D), jnp.float32),
    pltpu.SemaphoreType.DMA,
]
_MESH = plsc.VectorSubcoreMesh(core_axis_name="c", subcore_axis_name="s")
_SC_OUT = jax.ShapeDtypeStruct((NC * NPAD, D), jnp.float32)


@functools.partial(pl.kernel, out_type=_SC_OUT, mesh=_MESH,
                   scratch_types=_SC_SCRATCH, name="sc_segsum")
def _sc_segsum(y_hbm, src_hbm, dst_hbm, zacc_hbm, lin_hbm, out_hbm,
               acc_sh, idx_s, idx_d, idx_lin, rows_v, sem):
  c = lax.axis_index("c")
  s = lax.axis_index("s")

  pltpu.sync_copy(zacc_hbm, rows_v)
  _zero_acc(acc_sh, rows_v, idx_lin, lin_hbm, s)
  plsc.subcore_barrier()

  ebase = (c * NS + s) * PT

  def step(i, carry):
    base = ebase + i * CHUNK
    pltpu.sync_copy(src_hbm.at[pl.ds(base, CHUNK)], idx_s)
    pltpu.sync_copy(dst_hbm.at[pl.ds(base, CHUNK)], idx_d)
    pltpu.async_copy(y_hbm.at[idx_s], rows_v, sem).wait()
    pltpu.sync_copy(rows_v, acc_sh.at[idx_d], add=True)
    return carry

  lax.fori_loop(0, STEPS, step, 0)
  plsc.subcore_barrier()
  _copy_out(acc_sh, rows_v, idx_lin, lin_hbm, out_hbm, c, s, sem)


@functools.partial(pl.kernel, out_type=_SC_OUT, mesh=_MESH,
                   scratch_types=_SC_SCRATCH, name="sc_degcnt")
def _sc_degcnt(dst_hbm, zacc_hbm, ones_hbm, lin_hbm, out_hbm,
               acc_sh, idx_s, idx_d, idx_lin, rows_v, sem):
  c = lax.axis_index("c")
  s = lax.axis_index("s")

  pltpu.sync_copy(zacc_hbm, rows_v)
  _zero_acc(acc_sh, rows_v, idx_lin, lin_hbm, s)
  pltpu.sync_copy(ones_hbm, rows_v)
  plsc.subcore_barrier()

  ebase = (c * NS + s) * PT

  def step(i, carry):
    base = ebase + i * CHUNK
    pltpu.sync_copy(dst_hbm.at[pl.ds(base, CHUNK)], idx_d)
    pltpu.sync_copy(rows_v, acc_sh.at[idx_d], add=True)
    return carry

  lax.fori_loop(0, STEPS, step, 0)
  plsc.subcore_barrier()
  _copy_out(acc_sh, rows_v, idx_lin, lin_hbm, out_hbm, c, s, sem)


BM = 1000


def _t1_body(x_ref, wl_ref, wr_ref, b_ref, y_ref, r_ref):
  xb = x_ref[...]
  y_ref[...] = jnp.dot(xb, wl_ref[...], preferred_element_type=jnp.float32)
  r_ref[...] = (jnp.dot(xb, wr_ref[...], preferred_element_type=jnp.float32)
                + b_ref[...])


_t1 = pl.pallas_call(
    _t1_body,
    grid=(N // BM,),
    in_specs=[
        pl.BlockSpec((BM, D), lambda i: (i, 0)),
        pl.BlockSpec((D, D), lambda i: (0, 0)),
        pl.BlockSpec((D, D), lambda i: (0, 0)),
        pl.BlockSpec((1, D), lambda i: (0, 0)),
    ],
    out_specs=[pl.BlockSpec((BM, D), lambda i: (i, 0))] * 2,
    out_shape=[jax.ShapeDtypeStruct((N, D), jnp.float32)] * 2,
)


def _t2_body(p_ref, dp_ref, r1_ref, wl_ref, wr_ref, b_ref, y_ref, r_ref):
  deg = jnp.maximum(dp_ref[0][:, 0:1] + dp_ref[1][:, 0:1], 1.0)
  h = jnp.maximum((p_ref[0] + p_ref[1]) / deg + r1_ref[...], 0.0)
  y_ref[...] = jnp.dot(h, wl_ref[...], preferred_element_type=jnp.float32)
  r_ref[...] = (jnp.dot(h, wr_ref[...], preferred_element_type=jnp.float32)
                + b_ref[...])


_t2 = pl.pallas_call(
    _t2_body,
    grid=(N // BM,),
    in_specs=[
        pl.BlockSpec((NC, BM, D), lambda i: (0, i, 0)),
        pl.BlockSpec((NC, BM, D), lambda i: (0, i, 0)),
        pl.BlockSpec((BM, D), lambda i: (i, 0)),
        pl.BlockSpec((D, D), lambda i: (0, 0)),
        pl.BlockSpec((D, D), lambda i: (0, 0)),
        pl.BlockSpec((1, D), lambda i: (0, 0)),
    ],
    out_specs=[pl.BlockSpec((BM, D), lambda i: (i, 0))] * 2,
    out_shape=[jax.ShapeDtypeStruct((N, D), jnp.float32)] * 2,
)


def _t3_body(q_ref, dp_ref, r2_ref, z_ref):
  deg = jnp.maximum(dp_ref[0][:, 0:1] + dp_ref[1][:, 0:1], 1.0)
  z_ref[...] = jnp.maximum((q_ref[0] + q_ref[1]) / deg + r2_ref[...], 0.0)


_t3 = pl.pallas_call(
    _t3_body,
    grid=(N // BM,),
    in_specs=[
        pl.BlockSpec((NC, BM, D), lambda i: (0, i, 0)),
        pl.BlockSpec((NC, BM, D), lambda i: (0, i, 0)),
        pl.BlockSpec((BM, D), lambda i: (i, 0)),
    ],
    out_specs=pl.BlockSpec((BM, D), lambda i: (i, 0)),
    out_shape=jax.ShapeDtypeStruct((N, D), jnp.float32),
)


@jax.jit
def kernel(x, edge_index, W1_l, b1_l, W1_r, W2_l, b2_l, W2_r):
  ei = edge_index.astype(jnp.int32)
  pad = EPAD - E
  src = jnp.concatenate([ei[0], jnp.zeros((pad,), jnp.int32)])
  dst = jnp.concatenate([ei[1], jnp.full((pad,), N, jnp.int32)])
  zacc = jnp.zeros((CHUNK, D), jnp.float32)
  ones_rows = jnp.ones((CHUNK, D), jnp.float32)
  lin = jnp.arange(NPAD, dtype=jnp.int32)

  y1, r1 = _t1(x, W1_l, W1_r, b1_l.reshape(1, D))
  dp = _sc_degcnt(dst, zacc, ones_rows, lin).reshape(NC, NPAD, D)
  p = _sc_segsum(y1, src, dst, zacc, lin).reshape(NC, NPAD, D)
  y2, r2 = _t2(p, dp, r1, W2_l, W2_r, b2_l.reshape(1, D))
  q = _sc_segsum(y2, src, dst, zacc, lin).reshape(NC, NPAD, D)
  z = _t3(q, dp, r2)
  return z


# speedup vs baseline: 3.2297x; 1.3840x over previous
import functools

import jax
import jax.numpy as jnp
from jax import lax
from jax.experimental import pallas as pl
from jax.experimental.pallas import tpu as pltpu
from jax.experimental.pallas import tpu_sc as plsc

N = 10000
E = 320000
D = 128
NC = 2
NS = 16
CHUNK = 128
PT = 10240
STEPS = PT // CHUNK
EPAD = NC * NS * PT
NPAD = 10240
STRIPE = NPAD // NS


def _zero_acc(acc_sh, rows_v, idx_lin, lin_hbm, s):
  for j in range(STRIPE // CHUNK):
    pltpu.sync_copy(lin_hbm.at[pl.ds(s * STRIPE + j * CHUNK, CHUNK)], idx_lin)
    pltpu.sync_copy(rows_v, acc_sh.at[idx_lin])


def _copy_out(acc_sh, rows_v, idx_lin, lin_hbm, out_hbm, c, s, sem):
  for j in range(STRIPE // CHUNK):
    r0 = s * STRIPE + j * CHUNK
    pltpu.sync_copy(lin_hbm.at[pl.ds(r0, CHUNK)], idx_lin)
    pltpu.async_copy(acc_sh.at[idx_lin], rows_v, sem).wait()
    pltpu.sync_copy(rows_v, out_hbm.at[pl.ds(c * NPAD + r0, CHUNK)])


_SC_SCRATCH = [
    pltpu.VMEM_SHARED((NPAD, D), jnp.float32),
    pltpu.VMEM((2, 2, CHUNK), jnp.int32),
    pltpu.VMEM((CHUNK,), jnp.int32),
    pltpu.VMEM((2, CHUNK, D), jnp.float32),
    pltpu.SemaphoreType.DMA,
    pltpu.SemaphoreType.DMA,
    pltpu.SemaphoreType.DMA,
]
_MESH = plsc.VectorSubcoreMesh(core_axis_name="c", subcore_axis_name="s")
_SC_OUT = jax.ShapeDtypeStruct((NC * NPAD, D), jnp.float32)


@functools.partial(pl.kernel, out_type=_SC_OUT, mesh=_MESH,
                   scratch_types=_SC_SCRATCH, name="sc_segsum")
def _sc_segsum(y_hbm, edges_hbm, zacc_hbm, lin_hbm, out_hbm,
               acc_sh, idx2, idx_lin, rows2, sem_a, sem_b, sem_i):
  c = lax.axis_index("c")
  s = lax.axis_index("s")
  estep = (c * NS + s) * STEPS

  pltpu.sync_copy(zacc_hbm, rows2.at[0])
  _zero_acc(acc_sh, rows2.at[0], idx_lin, lin_hbm, s)
  plsc.subcore_barrier()

  pltpu.sync_copy(edges_hbm.at[estep], idx2.at[0])
  pltpu.async_copy(y_hbm.at[idx2.at[0, 0]], rows2.at[0], sem_a)
  pltpu.async_copy(edges_hbm.at[estep + 1], idx2.at[1], sem_i)

  def pair(k2, carry):
    i = 2 * k2
    pltpu.make_async_copy(y_hbm.at[idx2.at[0, 0]], rows2.at[0], sem_a).wait()
    pltpu.make_async_copy(edges_hbm.at[estep + i + 1], idx2.at[1],
                          sem_i).wait()
    pltpu.async_copy(y_hbm.at[idx2.at[1, 0]], rows2.at[1], sem_b)
    pltpu.sync_copy(rows2.at[0], acc_sh.at[idx2.at[0, 1]], add=True)

    @pl.when(i + 2 < STEPS)
    def _():
      pltpu.async_copy(edges_hbm.at[estep + i + 2], idx2.at[0], sem_i)

    pltpu.make_async_copy(y_hbm.at[idx2.at[1, 0]], rows2.at[1], sem_b).wait()

    @pl.when(i + 2 < STEPS)
    def _():
      pltpu.make_async_copy(edges_hbm.at[estep + i + 2], idx2.at[0],
                            sem_i).wait()
      pltpu.async_copy(y_hbm.at[idx2.at[0, 0]], rows2.at[0], sem_a)

    pltpu.sync_copy(rows2.at[1], acc_sh.at[idx2.at[1, 1]], add=True)

    @pl.when(i + 3 < STEPS)
    def _():
      pltpu.async_copy(edges_hbm.at[estep + i + 3], idx2.at[1], sem_i)

    return carry

  lax.fori_loop(0, STEPS // 2, pair, 0)
  plsc.subcore_barrier()
  _copy_out(acc_sh, rows2.at[0], idx_lin, lin_hbm, out_hbm, c, s, sem_a)


@functools.partial(pl.kernel, out_type=_SC_OUT, mesh=_MESH,
                   scratch_types=_SC_SCRATCH, name="sc_degcnt")
def _sc_degcnt(edges_hbm, zacc_hbm, ones_hbm, lin_hbm, out_hbm,
               acc_sh, idx2, idx_lin, rows2, sem_a, sem_b, sem_i):
  c = lax.axis_index("c")
  s = lax.axis_index("s")
  estep = (c * NS + s) * STEPS

  pltpu.sync_copy(zacc_hbm, rows2.at[0])
  _zero_acc(acc_sh, rows2.at[0], idx_lin, lin_hbm, s)
  pltpu.sync_copy(ones_hbm, rows2.at[0])
  plsc.subcore_barrier()

  pltpu.sync_copy(edges_hbm.at[estep], idx2.at[0])
  pltpu.async_copy(edges_hbm.at[estep + 1], idx2.at[1], sem_i)

  def pair(k2, carry):
    i = 2 * k2
    pltpu.sync_copy(rows2.at[0], acc_sh.at[idx2.at[0, 1]], add=True)
    pltpu.make_async_copy(edges_hbm.at[estep + i + 1], idx2.at[1],
                          sem_i).wait()

    @pl.when(i + 2 < STEPS)
    def _():
      pltpu.async_copy(edges_hbm.at[estep + i + 2], idx2.at[0], sem_i)

    pltpu.sync_copy(rows2.at[0], acc_sh.at[idx2.at[1, 1]], add=True)

    @pl.when(i + 2 < STEPS)
    def _():
      pltpu.make_async_copy(edges_hbm.at[estep + i + 2], idx2.at[0],
                            sem_i).wait()

    @pl.when(i + 3 < STEPS)
    def _():
      pltpu.async_copy(edges_hbm.at[estep + i + 3], idx2.at[1], sem_i)

    return carry

  lax.fori_loop(0, STEPS // 2, pair, 0)
  plsc.subcore_barrier()
  _copy_out(acc_sh, rows2.at[0], idx_lin, lin_hbm, out_hbm, c, s, sem_a)


BM = 1000


def _t1_body(x_ref, wl_ref, wr_ref, b_ref, y_ref, r_ref):
  xb = x_ref[...]
  y_ref[...] = jnp.dot(xb, wl_ref[...], preferred_element_type=jnp.float32)
  r_ref[...] = (jnp.dot(xb, wr_ref[...], preferred_element_type=jnp.float32)
                + b_ref[...])


_t1 = pl.pallas_call(
    _t1_body,
    grid=(N // BM,),
    in_specs=[
        pl.BlockSpec((BM, D), lambda i: (i, 0)),
        pl.BlockSpec((D, D), lambda i: (0, 0)),
        pl.BlockSpec((D, D), lambda i: (0, 0)),
        pl.BlockSpec((1, D), lambda i: (0, 0)),
    ],
    out_specs=[pl.BlockSpec((BM, D), lambda i: (i, 0))] * 2,
    out_shape=[jax.ShapeDtypeStruct((N, D), jnp.float32)] * 2,
)


def _t2_body(p_ref, dp_ref, r1_ref, wl_ref, wr_ref, b_ref, y_ref, r_ref):
  deg = jnp.maximum(dp_ref[0][:, 0:1] + dp_ref[1][:, 0:1], 1.0)
  h = jnp.maximum((p_ref[0] + p_ref[1]) / deg + r1_ref[...], 0.0)
  y_ref[...] = jnp.dot(h, wl_ref[...], preferred_element_type=jnp.float32)
  r_ref[...] = (jnp.dot(h, wr_ref[...], preferred_element_type=jnp.float32)
                + b_ref[...])


_t2 = pl.pallas_call(
    _t2_body,
    grid=(N // BM,),
    in_specs=[
        pl.BlockSpec((NC, BM, D), lambda i: (0, i, 0)),
        pl.BlockSpec((NC, BM, D), lambda i: (0, i, 0)),
        pl.BlockSpec((BM, D), lambda i: (i, 0)),
        pl.BlockSpec((D, D), lambda i: (0, 0)),
        pl.BlockSpec((D, D), lambda i: (0, 0)),
        pl.BlockSpec((1, D), lambda i: (0, 0)),
    ],
    out_specs=[pl.BlockSpec((BM, D), lambda i: (i, 0))] * 2,
    out_shape=[jax.ShapeDtypeStruct((N, D), jnp.float32)] * 2,
)


def _t3_body(q_ref, dp_ref, r2_ref, z_ref):
  deg = jnp.maximum(dp_ref[0][:, 0:1] + dp_ref[1][:, 0:1], 1.0)
  z_ref[...] = jnp.maximum((q_ref[0] + q_ref[1]) / deg + r2_ref[...], 0.0)


_t3 = pl.pallas_call(
    _t3_body,
    grid=(N // BM,),
    in_specs=[
        pl.BlockSpec((NC, BM, D), lambda i: (0, i, 0)),
        pl.BlockSpec((NC, BM, D), lambda i: (0, i, 0)),
        pl.BlockSpec((BM, D), lambda i: (i, 0)),
    ],
    out_specs=pl.BlockSpec((BM, D), lambda i: (i, 0)),
    out_shape=jax.ShapeDtypeStruct((N, D), jnp.float32),
)


@jax.jit
def kernel(x, edge_index, W1_l, b1_l, W1_r, W2_l, b2_l, W2_r):
  ei = edge_index.astype(jnp.int32)
  pad = EPAD - E
  src = jnp.concatenate([ei[0], jnp.zeros((pad,), jnp.int32)])
  dst = jnp.concatenate([ei[1], jnp.full((pad,), N, jnp.int32)])
  edges = jnp.stack(
      [src.reshape(-1, CHUNK), dst.reshape(-1, CHUNK)], axis=1)
  zacc = jnp.zeros((CHUNK, D), jnp.float32)
  ones_rows = jnp.ones((CHUNK, D), jnp.float32)
  lin = jnp.arange(NPAD, dtype=jnp.int32)

  y1, r1 = _t1(x, W1_l, W1_r, b1_l.reshape(1, D))
  dp = _sc_degcnt(edges, zacc, ones_rows, lin).reshape(NC, NPAD, D)
  p = _sc_segsum(y1, edges, zacc, lin).reshape(NC, NPAD, D)
  y2, r2 = _t2(p, dp, r1, W2_l, W2_r, b2_l.reshape(1, D))
  q = _sc_segsum(y2, edges, zacc, lin).reshape(NC, NPAD, D)
  z = _t3(q, dp, r2)
  return z


# speedup vs baseline: 3.8517x; 1.1926x over previous
import functools

import jax
import jax.numpy as jnp
from jax import lax
from jax.experimental import pallas as pl
from jax.experimental.pallas import tpu as pltpu
from jax.experimental.pallas import tpu_sc as plsc

N = 10000
E = 320000
D = 128
NC = 2
NS = 16
CHUNK = 128
PT = 10240
STEPS = PT // CHUNK
EPAD = NC * NS * PT
NPAD = 10240
STRIPE = NPAD // NS
F0 = 116
F1 = 2 * STEPS - F0


def _zero_acc(acc_sh, rows_v, idx_lin, lin_hbm, s):
  for j in range(STRIPE // CHUNK):
    pltpu.sync_copy(lin_hbm.at[pl.ds(s * STRIPE + j * CHUNK, CHUNK)], idx_lin)
    pltpu.sync_copy(rows_v, acc_sh.at[idx_lin])


def _copy_out(acc_sh, rows_v, idx_lin, lin_hbm, out_hbm, c, s, sem):
  for j in range(STRIPE // CHUNK):
    r0 = s * STRIPE + j * CHUNK
    pltpu.sync_copy(lin_hbm.at[pl.ds(r0, CHUNK)], idx_lin)
    pltpu.async_copy(acc_sh.at[idx_lin], rows_v, sem).wait()
    pltpu.sync_copy(rows_v, out_hbm.at[pl.ds(c * NPAD + r0, CHUNK)])


_SC_SCRATCH = [
    pltpu.VMEM_SHARED((NPAD, D), jnp.float32),
    pltpu.VMEM((2, 2, CHUNK), jnp.int32),
    pltpu.VMEM((CHUNK,), jnp.int32),
    pltpu.VMEM((2, CHUNK, D), jnp.float32),
    pltpu.SemaphoreType.DMA,
    pltpu.SemaphoreType.DMA,
    pltpu.SemaphoreType.DMA,
]
_MESH = plsc.VectorSubcoreMesh(core_axis_name="c", subcore_axis_name="s")
_SC_OUT = jax.ShapeDtypeStruct((NC * NPAD, D), jnp.float32)


@functools.partial(pl.kernel, out_type=_SC_OUT, mesh=_MESH,
                   scratch_types=_SC_SCRATCH, name="sc_segsum")
def _sc_segsum(y_hbm, edges_hbm, zacc_hbm, lin_hbm, out_hbm,
               acc_sh, idx2, idx_lin, rows2, sem_a, sem_b, sem_i):
  c = lax.axis_index("c")
  s = lax.axis_index("s")
  estep = jnp.where(c == 0, s * F0, NS * F0 + s * F1)
  nsteps = jnp.where(c == 0, F0, F1)

  pltpu.sync_copy(zacc_hbm, rows2.at[0])
  _zero_acc(acc_sh, rows2.at[0], idx_lin, lin_hbm, s)
  plsc.subcore_barrier()

  pltpu.sync_copy(edges_hbm.at[estep], idx2.at[0])
  pltpu.async_copy(y_hbm.at[idx2.at[0, 0]], rows2.at[0], sem_a)
  pltpu.async_copy(edges_hbm.at[estep + 1], idx2.at[1], sem_i)

  def pair(k2, carry):
    i = 2 * k2
    pltpu.make_async_copy(y_hbm.at[idx2.at[0, 0]], rows2.at[0], sem_a).wait()
    pltpu.make_async_copy(edges_hbm.at[estep + i + 1], idx2.at[1],
                          sem_i).wait()
    pltpu.async_copy(y_hbm.at[idx2.at[1, 0]], rows2.at[1], sem_b)
    pltpu.sync_copy(rows2.at[0], acc_sh.at[idx2.at[0, 1]], add=True)

    @pl.when(i + 2 < nsteps)
    def _():
      pltpu.async_copy(edges_hbm.at[estep + i + 2], idx2.at[0], sem_i)

    pltpu.make_async_copy(y_hbm.at[idx2.at[1, 0]], rows2.at[1], sem_b).wait()

    @pl.when(i + 2 < nsteps)
    def _():
      pltpu.make_async_copy(edges_hbm.at[estep + i + 2], idx2.at[0],
                            sem_i).wait()
      pltpu.async_copy(y_hbm.at[idx2.at[0, 0]], rows2.at[0], sem_a)

    pltpu.sync_copy(rows2.at[1], acc_sh.at[idx2.at[1, 1]], add=True)

    @pl.when(i + 3 < nsteps)
    def _():
      pltpu.async_copy(edges_hbm.at[estep + i + 3], idx2.at[1], sem_i)

    return carry

  lax.fori_loop(0, nsteps // 2, pair, 0)
  plsc.subcore_barrier()
  _copy_out(acc_sh, rows2.at[0], idx_lin, lin_hbm, out_hbm, c, s, sem_a)


@functools.partial(pl.kernel, out_type=_SC_OUT, mesh=_MESH,
                   scratch_types=_SC_SCRATCH, name="sc_degcnt")
def _sc_degcnt(edges_hbm, zacc_hbm, ones_hbm, lin_hbm, out_hbm,
               acc_sh, idx2, idx_lin, rows2, sem_a, sem_b, sem_i):
  c = lax.axis_index("c")
  s = lax.axis_index("s")
  estep = (c * NS + s) * STEPS

  pltpu.sync_copy(zacc_hbm, rows2.at[0])
  _zero_acc(acc_sh, rows2.at[0], idx_lin, lin_hbm, s)
  pltpu.sync_copy(ones_hbm, rows2.at[0])
  plsc.subcore_barrier()

  pltpu.sync_copy(edges_hbm.at[estep], idx2.at[0])
  pltpu.async_copy(edges_hbm.at[estep + 1], idx2.at[1], sem_i)

  def pair(k2, carry):
    i = 2 * k2
    pltpu.sync_copy(rows2.at[0], acc_sh.at[idx2.at[0, 1]], add=True)
    pltpu.make_async_copy(edges_hbm.at[estep + i + 1], idx2.at[1],
                          sem_i).wait()

    @pl.when(i + 2 < STEPS)
    def _():
      pltpu.async_copy(edges_hbm.at[estep + i + 2], idx2.at[0], sem_i)

    pltpu.sync_copy(rows2.at[0], acc_sh.at[idx2.at[1, 1]], add=True)

    @pl.when(i + 2 < STEPS)
    def _():
      pltpu.make_async_copy(edges_hbm.at[estep + i + 2], idx2.at[0],
                            sem_i).wait()

    @pl.when(i + 3 < STEPS)
    def _():
      pltpu.async_copy(edges_hbm.at[estep + i + 3], idx2.at[1], sem_i)

    return carry

  lax.fori_loop(0, STEPS // 2, pair, 0)
  plsc.subcore_barrier()
  _copy_out(acc_sh, rows2.at[0], idx_lin, lin_hbm, out_hbm, c, s, sem_a)


BM = 1000


def _t1_body(x_ref, wl_ref, wr_ref, b_ref, y_ref, r_ref):
  xb = x_ref[...]
  y_ref[...] = jnp.dot(xb, wl_ref[...], preferred_element_type=jnp.float32)
  r_ref[...] = (jnp.dot(xb, wr_ref[...], preferred_element_type=jnp.float32)
                + b_ref[...])


_t1 = pl.pallas_call(
    _t1_body,
    grid=(N // BM,),
    in_specs=[
        pl.BlockSpec((BM, D), lambda i: (i, 0)),
        pl.BlockSpec((D, D), lambda i: (0, 0)),
        pl.BlockSpec((D, D), lambda i: (0, 0)),
        pl.BlockSpec((1, D), lambda i: (0, 0)),
    ],
    out_specs=[pl.BlockSpec((BM, D), lambda i: (i, 0))] * 2,
    out_shape=[jax.ShapeDtypeStruct((N, D), jnp.float32)] * 2,
)


def _t2_body(p_ref, dp_ref, r1_ref, wl_ref, wr_ref, b_ref, y_ref, r_ref):
  deg = jnp.maximum(dp_ref[0][:, 0:1] + dp_ref[1][:, 0:1], 1.0)
  h = jnp.maximum((p_ref[0] + p_ref[1]) / deg + r1_ref[...], 0.0)
  y_ref[...] = jnp.dot(h, wl_ref[...], preferred_element_type=jnp.float32)
  r_ref[...] = (jnp.dot(h, wr_ref[...], preferred_element_type=jnp.float32)
                + b_ref[...])


_t2 = pl.pallas_call(
    _t2_body,
    grid=(N // BM,),
    in_specs=[
        pl.BlockSpec((NC, BM, D), lambda i: (0, i, 0)),
        pl.BlockSpec((NC, BM, D), lambda i: (0, i, 0)),
        pl.BlockSpec((BM, D), lambda i: (i, 0)),
        pl.BlockSpec((D, D), lambda i: (0, 0)),
        pl.BlockSpec((D, D), lambda i: (0, 0)),
        pl.BlockSpec((1, D), lambda i: (0, 0)),
    ],
    out_specs=[pl.BlockSpec((BM, D), lambda i: (i, 0))] * 2,
    out_shape=[jax.ShapeDtypeStruct((N, D), jnp.float32)] * 2,
)


def _t3_body(q_ref, dp_ref, r2_ref, z_ref):
  deg = jnp.maximum(dp_ref[0][:, 0:1] + dp_ref[1][:, 0:1], 1.0)
  z_ref[...] = jnp.maximum((q_ref[0] + q_ref[1]) / deg + r2_ref[...], 0.0)


_t3 = pl.pallas_call(
    _t3_body,
    grid=(N // BM,),
    in_specs=[
        pl.BlockSpec((NC, BM, D), lambda i: (0, i, 0)),
        pl.BlockSpec((NC, BM, D), lambda i: (0, i, 0)),
        pl.BlockSpec((BM, D), lambda i: (i, 0)),
    ],
    out_specs=pl.BlockSpec((BM, D), lambda i: (i, 0)),
    out_shape=jax.ShapeDtypeStruct((N, D), jnp.float32),
)


@jax.jit
def kernel(x, edge_index, W1_l, b1_l, W1_r, W2_l, b2_l, W2_r):
  ei = edge_index.astype(jnp.int32)
  pad = EPAD - E
  src = jnp.concatenate([ei[0], jnp.zeros((pad,), jnp.int32)])
  dst = jnp.concatenate([ei[1], jnp.full((pad,), N, jnp.int32)])
  edges = jnp.stack(
      [src.reshape(-1, CHUNK), dst.reshape(-1, CHUNK)], axis=1)
  zacc = jnp.zeros((CHUNK, D), jnp.float32)
  ones_rows = jnp.ones((CHUNK, D), jnp.float32)
  lin = jnp.arange(NPAD, dtype=jnp.int32)

  y1, r1 = _t1(x, W1_l, W1_r, b1_l.reshape(1, D))
  dp = _sc_degcnt(edges, zacc, ones_rows, lin).reshape(NC, NPAD, D)
  p = _sc_segsum(y1, edges, zacc, lin).reshape(NC, NPAD, D)
  y2, r2 = _t2(p, dp, r1, W2_l, W2_r, b2_l.reshape(1, D))
  q = _sc_segsum(y2, edges, zacc, lin).reshape(NC, NPAD, D)
  z = _t3(q, dp, r2)
  return z
